# baseline (speedup 1.0000x reference)
import functools

import jax
import jax.numpy as jnp
from jax import lax
from jax.experimental import pallas as pl
from jax.experimental.pallas import tpu as pltpu
from jax.experimental.pallas import tpu_sc as plsc

N = 10000
E = 320000
H = 128
DT = 0.03

GW = 128
BE = 1280
RC = 400
NCH = N // RC

_MESH = dict(core_axis_name="core", subcore_axis_name="subcore")


def _sc_gather(h, idx_flat):
    M = idx_flat.shape[1]
    mesh = plsc.VectorSubcoreMesh(**_MESH)

    @functools.partial(
        pl.kernel,
        out_type=jax.ShapeDtypeStruct((M, H), jnp.float32),
        mesh=mesh,
    )
    def k(h_hbm, i_hbm, o_hbm):
        def body(i_vmem, o_vmem):
            pltpu.sync_copy(h_hbm.at[i_vmem.at[0]], o_vmem)

        pltpu.emit_pipeline(
            body,
            grid=(M // GW,),
            in_specs=[pl.BlockSpec((1, GW), lambda i: (0, i))],
            out_specs=[pl.BlockSpec((GW, H), lambda i: (i, 0))],
            core_axis_name=("core", "subcore"),
            dimension_semantics=(pltpu.PARALLEL,),
        )(i_hbm, o_hbm)

    return k(h, idx_flat)


def _tc_mlp(gath, W1, b1, w2v, b2v, W2bd, b2pf):
    nblk = E // BE

    def body(xr_ref, xc_ref, W1_ref, b1_ref, w2v_ref, b2v_ref, W2_ref,
             b2pf_ref, o_ref):
        xr = xr_ref[...]
        xc = xc_ref[...]
        x = jnp.concatenate([xr, xc], axis=1).astype(jnp.bfloat16)
        z = jnp.dot(x, W1_ref[...],
                    preferred_element_type=jnp.float32) + b1_ref[...]
        t_vp = jnp.tanh(z[:, :256])
        t_f = jnp.maximum(z[:, 256:], 0.0)
        nu = (jnp.sum(t_vp[:, :128] * w2v_ref[...], axis=1, keepdims=True)
              + b2v_ref[0, 0])
        tpf = jnp.concatenate([t_vp[:, 128:], t_f],
                              axis=1).astype(jnp.bfloat16)
        pf = jnp.dot(tpf, W2_ref[...],
                     preferred_element_type=jnp.float32) + b2pf_ref[...]
        o_ref[...] = nu * (xc - xr) + pf[:, 128:] - pf[:, :128]

    full = lambda shape: pl.BlockSpec(shape, lambda i: (0, 0))
    return pl.pallas_call(
        body,
        grid=(nblk,),
        in_specs=[
            pl.BlockSpec((BE, H), lambda i: (i, 0)),
            pl.BlockSpec((BE, H), lambda i: (i + nblk, 0)),
            full((256, 384)),
            full((1, 384)),
            full((1, 128)),
            full((1, 1)),
            full((256, 256)),
            full((1, 256)),
        ],
        out_specs=pl.BlockSpec((BE, H), lambda i: (i, 0)),
        out_shape=jax.ShapeDtypeStruct((E, H), jnp.float32),
    )(gath, gath, W1, b1, w2v, b2v, W2bd, b2pf)


def _sc_scatter(msgs, row_flat, zrows):
    mesh = plsc.VectorSubcoreMesh(**_MESH)

    @functools.partial(
        pl.kernel,
        out_type=jax.ShapeDtypeStruct((2, N, H), jnp.float32),
        mesh=mesh,
        scratch_types=[pltpu.VMEM_SHARED((N, H), jnp.float32)],
    )
    def k(m_hbm, i_hbm, z_hbm, o_hbm, acc):
        sid = lax.axis_index("subcore")
        cid = lax.axis_index("core")

        @pl.loop(0, 2)
        def _(j):
            c = j * 16 + sid

            @pl.when(c < NCH)
            def _():
                pltpu.sync_copy(z_hbm, acc.at[pl.ds(c * RC, RC), :])

        plsc.subcore_barrier()

        def body(m_vmem, i_vmem):
            pltpu.sync_copy(m_vmem, acc.at[i_vmem.at[0]], add=True)

        pltpu.emit_pipeline(
            body,
            grid=(E // GW,),
            in_specs=[
                pl.BlockSpec((GW, H), lambda i: (i, 0)),
                pl.BlockSpec((1, GW), lambda i: (0, i)),
            ],
            out_specs=[],
            core_axis_name=("core", "subcore"),
            dimension_semantics=(pltpu.PARALLEL,),
        )(m_hbm, i_hbm)
        plsc.subcore_barrier()

        @pl.loop(0, 2)
        def _(j):
            c = j * 16 + sid

            @pl.when(c < NCH)
            def _():
                pltpu.sync_copy(acc.at[pl.ds(c * RC, RC), :],
                                o_hbm.at[cid, pl.ds(c * RC, RC), :])

    return k(msgs, row_flat, zrows)


def _tc_combine(h, parts):
    def body(h_ref, p_ref, o_ref):
        o_ref[...] = h_ref[...] + DT * (p_ref[0] + p_ref[1])

    return pl.pallas_call(
        body,
        grid=(10,),
        in_specs=[
            pl.BlockSpec((N // 10, H), lambda i: (i, 0)),
            pl.BlockSpec((2, N // 10, H), lambda i: (0, i, 0)),
        ],
        out_specs=pl.BlockSpec((N // 10, H), lambda i: (i, 0)),
        out_shape=jax.ShapeDtypeStruct((N, H), jnp.float32),
    )(h, parts)


def kernel(h, edge_index, v_w1, v_b1, v_w2, v_b2, p_w1, p_b1, p_w2, p_b2,
           f_w1, f_b1, f_w2, f_b2):
    idx_flat = edge_index.reshape(1, 2 * E)
    row_flat = edge_index[0:1, :]

    W1 = jnp.concatenate([v_w1.T, p_w1.T, f_w1.T],
                         axis=1).astype(jnp.bfloat16)
    b1 = jnp.concatenate([v_b1, p_b1, f_b1]).reshape(1, 384)
    W2bd = jnp.zeros((256, 256), jnp.float32)
    W2bd = W2bd.at[:128, :128].set(p_w2.T).at[128:, 128:].set(f_w2.T)
    W2bd = W2bd.astype(jnp.bfloat16)
    b2pf = jnp.concatenate([p_b2, f_b2]).reshape(1, 256)
    w2v = v_w2.reshape(1, 128)
    b2v = v_b2.reshape(1, 1)
    zrows = jnp.zeros((RC, H), jnp.float32)

    gath = _sc_gather(h, idx_flat)
    msgs = _tc_mlp(gath, W1, b1, w2v, b2v, W2bd, b2pf)
    parts = _sc_scatter(msgs, row_flat, zrows)
    return _tc_combine(h, parts)


# speedup vs baseline: 1.2254x; 1.2254x over previous
import functools

import jax
import jax.numpy as jnp
from jax import lax
from jax.experimental import pallas as pl
from jax.experimental.pallas import tpu as pltpu
from jax.experimental.pallas import tpu_sc as plsc

N = 10000
E = 320000
H = 128
DT = 0.03

C = 4
EC = E // C
GW = 128
BE = 1600
RC = 400
NCH = N // RC

_MESH = dict(core_axis_name="core", subcore_axis_name="subcore")


def _sc_gather(h, idx_flat):
    M = idx_flat.shape[1]
    mesh = plsc.VectorSubcoreMesh(**_MESH)

    @functools.partial(
        pl.kernel,
        out_type=jax.ShapeDtypeStruct((M, H), jnp.float32),
        mesh=mesh,
    )
    def k(h_hbm, i_hbm, o_hbm):
        def body(i_vmem, o_vmem):
            pltpu.sync_copy(h_hbm.at[i_vmem.at[0]], o_vmem)

        pltpu.emit_pipeline(
            body,
            grid=(M // GW,),
            in_specs=[pl.BlockSpec((1, GW), lambda i: (0, i))],
            out_specs=[pl.BlockSpec((GW, H), lambda i: (i, 0))],
            core_axis_name=("core", "subcore"),
            dimension_semantics=(pltpu.PARALLEL,),
        )(i_hbm, o_hbm)

    return k(h, idx_flat)


def _tc_mlp(gath, W1, b1, w2v, b2v, W2bd, b2pf):
    nblk = EC // BE

    def body(xr_ref, xc_ref, W1_ref, b1_ref, w2v_ref, b2v_ref, W2_ref,
             b2pf_ref, o_ref):
        xr = xr_ref[...]
        xc = xc_ref[...]
        x = jnp.concatenate([xr, xc], axis=1).astype(jnp.bfloat16)
        z = jnp.dot(x, W1_ref[...],
                    preferred_element_type=jnp.float32) + b1_ref[...]
        t_vp = jnp.tanh(z[:, :256])
        t_f = jnp.maximum(z[:, 256:], 0.0)
        nu = (jnp.sum(t_vp[:, :128] * w2v_ref[...], axis=1, keepdims=True)
              + b2v_ref[0, 0])
        tpf = jnp.concatenate([t_vp[:, 128:], t_f],
                              axis=1).astype(jnp.bfloat16)
        pf = jnp.dot(tpf, W2_ref[...],
                     preferred_element_type=jnp.float32) + b2pf_ref[...]
        o_ref[...] = nu * (xc - xr) + pf[:, 128:] - pf[:, :128]

    full = lambda shape: pl.BlockSpec(shape, lambda i: (0, 0))
    return pl.pallas_call(
        body,
        grid=(nblk,),
        in_specs=[
            pl.BlockSpec((BE, H), lambda i: (i, 0)),
            pl.BlockSpec((BE, H), lambda i: (i + nblk, 0)),
            full((256, 384)),
            full((1, 384)),
            full((1, 128)),
            full((1, 1)),
            full((256, 256)),
            full((1, 256)),
        ],
        out_specs=pl.BlockSpec((BE, H), lambda i: (i, 0)),
        out_shape=jax.ShapeDtypeStruct((EC, H), jnp.float32),
    )(gath, gath, W1, b1, w2v, b2v, W2bd, b2pf)


def _sc_scatter(msg_chunks, idx_chunks, zrows):
    mesh = plsc.VectorSubcoreMesh(**_MESH)

    @functools.partial(
        pl.kernel,
        out_type=jax.ShapeDtypeStruct((2, N, H), jnp.float32),
        mesh=mesh,
        scratch_types=[pltpu.VMEM_SHARED((N, H), jnp.float32)],
    )
    def k(*refs):
        m_refs = refs[:C]
        i_refs = refs[C:2 * C]
        z_hbm = refs[2 * C]
        o_hbm = refs[2 * C + 1]
        acc = refs[2 * C + 2]
        sid = lax.axis_index("subcore")
        cid = lax.axis_index("core")

        @pl.loop(0, 2)
        def _(j):
            c = j * 16 + sid

            @pl.when(c < NCH)
            def _():
                pltpu.sync_copy(z_hbm, acc.at[pl.ds(c * RC, RC), :])

        plsc.subcore_barrier()

        def body(m_vmem, i_vmem):
            pltpu.sync_copy(m_vmem, acc.at[i_vmem.at[0]], add=True)

        for c in range(C):
            pltpu.emit_pipeline(
                body,
                grid=(EC // GW,),
                in_specs=[
                    pl.BlockSpec((GW, H), lambda i: (i, 0)),
                    pl.BlockSpec((1, GW), lambda i: (0, i)),
                ],
                out_specs=[],
                core_axis_name=("core", "subcore"),
                dimension_semantics=(pltpu.PARALLEL,),
            )(m_refs[c], i_refs[c])
        plsc.subcore_barrier()

        @pl.loop(0, 2)
        def _(j):
            c = j * 16 + sid

            @pl.when(c < NCH)
            def _():
                pltpu.sync_copy(acc.at[pl.ds(c * RC, RC), :],
                                o_hbm.at[cid, pl.ds(c * RC, RC), :])

    return k(*msg_chunks, *idx_chunks, zrows)


def _tc_combine(h, parts):
    def body(h_ref, p_ref, o_ref):
        o_ref[...] = h_ref[...] + DT * (p_ref[0] + p_ref[1])

    return pl.pallas_call(
        body,
        grid=(10,),
        in_specs=[
            pl.BlockSpec((N // 10, H), lambda i: (i, 0)),
            pl.BlockSpec((2, N // 10, H), lambda i: (0, i, 0)),
        ],
        out_specs=pl.BlockSpec((N // 10, H), lambda i: (i, 0)),
        out_shape=jax.ShapeDtypeStruct((N, H), jnp.float32),
    )(h, parts)


def kernel(h, edge_index, v_w1, v_b1, v_w2, v_b2, p_w1, p_b1, p_w2, p_b2,
           f_w1, f_b1, f_w2, f_b2):
    rowc = edge_index[0].reshape(C, EC)
    colc = edge_index[1].reshape(C, EC)
    idx_pair = jnp.stack([rowc, colc], axis=1)

    W1 = jnp.concatenate([v_w1.T, p_w1.T, f_w1.T],
                         axis=1).astype(jnp.bfloat16)
    b1 = jnp.concatenate([v_b1, p_b1, f_b1]).reshape(1, 384)
    W2bd = jnp.zeros((256, 256), jnp.float32)
    W2bd = W2bd.at[:128, :128].set(p_w2.T).at[128:, 128:].set(f_w2.T)
    W2bd = W2bd.astype(jnp.bfloat16)
    b2pf = jnp.concatenate([p_b2, f_b2]).reshape(1, 256)
    w2v = v_w2.reshape(1, 128)
    b2v = v_b2.reshape(1, 1)
    zrows = jnp.zeros((RC, H), jnp.float32)

    msg_chunks = []
    idx_chunks = []
    for c in range(C):
        gath = _sc_gather(h, idx_pair[c].reshape(1, 2 * EC))
        msg_chunks.append(_tc_mlp(gath, W1, b1, w2v, b2v, W2bd, b2pf))
        idx_chunks.append(rowc[c:c + 1, :])

    parts = _sc_scatter(msg_chunks, idx_chunks, zrows)
    return _tc_combine(h, parts)
